# baseline (speedup 1.0000x reference)
import functools

import jax
import jax.numpy as jnp
from jax import lax
from jax.experimental import pallas as pl
from jax.experimental.pallas import tpu as pltpu
from jax.experimental.pallas import tpu_sc as plsc

USER_N = 1000000
MOVIE_N = 100000
K = 32
B = 16384

NC = 2
NS = 16
NW = NC * NS
BPW = B // NW
NGRP = BPW // 16

CH = 512
NBUF = 2
NBK = 64
BK_U = 48
NBK_M = 10
BK_M = 160
SENTINEL = 0x3FFFFFFF

SLAB_U = 31232
NCH_U = SLAB_U // CH
U_TAIL = (NW * SLAB_U, 512)
U_PART = USER_N - USER_N % 128
U_PART_SZ = USER_N - U_PART

SLAB_M = 3072
NCH_M = SLAB_M // CH
M_TAILS = [(98304, 512), (98816, 512), (99328, 512), (99840, 128)]
M_PART = MOVIE_N - MOVIE_N % 128
M_PART_SZ = MOVIE_N - M_PART


def _extract_table(wid, sid, idx_hbm, tbl_hbm, tail_hbm, tail_buf, out1d,
                   idx_all, bufs, spbufs, sems, hit_uj, hit_j, stage, cnt,
                   osem, slab, n_total, nch, tails, part_lo, nbk, bk_cap):
    lane = lax.iota(jnp.int32, 16)
    lane0 = lane == 0
    lo = wid * slab
    hi = jnp.where(wid == NW - 1, n_total, lo + slab)

    pltpu.sync_copy(idx_hbm, idx_all)
    cnt[0] = 0
    cnt[1] = 0

    def zero_bk(i, carry):
        cnt[8 + i] = 0
        return carry
    lax.fori_loop(0, nbk, zero_bk, 0)

    def init(v, carry):
        hit_uj[pl.ds(v * 16, 16)] = jnp.full((16,), SENTINEL, jnp.int32)
        return carry
    lax.fori_loop(0, (nbk * bk_cap) // 16, init, 0)

    def scanv(v, carry):
        uv = idx_all[pl.ds(v * 16, 16)]
        m_init = ((uv >= lo) & (uv < hi)).astype(jnp.int32)

        @pl.when(jnp.max(m_init) > 0)
        def _():
            def bit_body(m):
                l = plsc.all_reduce_ffs(m != 0)
                one = (lane == l).astype(jnp.int32)
                uj_s = jnp.sum(uv * one)
                l_s = jnp.max(l)
                bk = (uj_s - lo) >> 9
                cb = cnt[8 + bk]
                slot = bk * bk_cap + cb
                svec = jnp.full((16,), slot, jnp.int32)
                plsc.store_scatter(hit_uj, [svec],
                                   jnp.full((16,), uj_s, jnp.int32),
                                   mask=lane0)
                plsc.store_scatter(hit_j, [svec],
                                   jnp.full((16,), v * 16 + l_s, jnp.int32),
                                   mask=lane0)
                cnt[8 + bk] = cb + 1
                return m * (1 - one)

            lax.while_loop(lambda m: jnp.max(m) > 0, bit_body, m_init)
        return carry

    lax.fori_loop(0, B // 16, scanv, 0)

    def process_chunk(buf, clo, sz):
        bk_base = ((clo - lo) >> 9) * bk_cap

        def hscan(h, carry):
            huj = hit_uj[pl.ds(bk_base + h * 16, 16)]
            hjv = hit_j[pl.ds(bk_base + h * 16, 16)]
            m_init = ((huj >= clo) & (huj < clo + sz)).astype(jnp.int32)

            @pl.when(jnp.max(m_init) > 0)
            def _():
                def bit_body(m):
                    l = plsc.all_reduce_ffs(m != 0)
                    one = (lane == l).astype(jnp.int32)
                    uj_s = jnp.sum(huj * one)
                    j_s = jnp.sum(hjv * one)
                    loc = jnp.full((16,), uj_s - clo, jnp.int32)
                    u0 = plsc.load_gather(buf, [lane, loc])
                    u1 = plsc.load_gather(buf, [lane + 16, loc])
                    c2 = cnt[1]
                    stage[pl.ds(c2 * K, 16)] = u0
                    stage[pl.ds(c2 * K + 16, 16)] = u1
                    pltpu.make_async_copy(
                        stage.at[pl.ds(c2 * K, K)],
                        out1d.at[pl.ds(j_s * K, K)],
                        osem).start()
                    cnt[1] = c2 + 1
                    return m * (1 - one)

                lax.while_loop(lambda m: jnp.max(m) > 0, bit_body, m_init)
            return carry

        lax.fori_loop(0, bk_cap // 16, hscan, 0)

    def sp_slice(b):
        return spbufs[b].at[:, pl.ds(sid * CH, CH)]

    def start_window(i, b):
        pltpu.make_async_copy(
            tbl_hbm.at[:, pl.ds(lo + i * CH, CH)], sp_slice(b),
            sems[b]).start()

    for b in range(NBUF):
        if b < nch:
            start_window(b, b)

    nloop = nch // NBUF

    def ring(g, carry):
        for b in range(NBUF):
            i = g * NBUF + b
            pltpu.make_async_copy(
                tbl_hbm.at[:, pl.ds(0, CH)], sp_slice(b), sems[b]).wait()
            pltpu.sync_copy(sp_slice(b), bufs[b])
            nxt = i + NBUF

            @pl.when(nxt < nch)
            def _():
                start_window(nxt, b)
            process_chunk(bufs[b], lo + i * CH, CH)
        return carry

    lax.fori_loop(0, nloop, ring, 0)

    for i in range(nloop * NBUF, nch):
        b = i % NBUF
        pltpu.make_async_copy(
            tbl_hbm.at[:, pl.ds(0, CH)], sp_slice(b), sems[b]).wait()
        pltpu.sync_copy(sp_slice(b), bufs[b])
        process_chunk(bufs[b], lo + i * CH, CH)

    @pl.when(wid == NW - 1)
    def _tail():
        for off, sz in tails:
            pltpu.sync_copy(tbl_hbm.at[:, pl.ds(jnp.int32(off), sz)],
                            bufs[0].at[:, pl.ds(0, sz)])
            process_chunk(bufs[0], jnp.int32(off), sz)
        pltpu.sync_copy(tail_hbm, tail_buf)
        process_chunk(tail_buf, jnp.int32(part_lo), tail_buf.shape[1])

    def drain(i, carry):
        pltpu.make_async_copy(
            out1d.at[pl.ds(0, K)], stage.at[pl.ds(0, K)], osem).wait()
        return carry
    lax.fori_loop(0, cnt[1], drain, 0)


def _extract_body(u_idx, m_idx, ut_hbm, mt_hbm, ut_tail, mt_tail,
                  u_rows, m_rows, idx_all, cb0, cb1, sp0, sp1,
                  hit_uj, hit_j, stage, tail_u_buf, tail_m_buf, cnt,
                  sem0, sem1, osem):
    sid = lax.axis_index("s")
    wid = sid * NC + lax.axis_index("c")
    bufs = (cb0, cb1)
    spbufs = (sp0, sp1)
    sems = (sem0, sem1)
    _extract_table(wid, sid, u_idx, ut_hbm, ut_tail, tail_u_buf, u_rows,
                   idx_all, bufs, spbufs, sems, hit_uj, hit_j, stage, cnt,
                   osem, SLAB_U, USER_N, NCH_U, [U_TAIL], U_PART, NBK, BK_U)
    _extract_table(wid, sid, m_idx, mt_hbm, mt_tail, tail_m_buf, m_rows,
                   idx_all, bufs, spbufs, sems, hit_uj, hit_j, stage, cnt,
                   osem, SLAB_M, MOVIE_N, NCH_M, M_TAILS, M_PART,
                   NBK_M, BK_M)


@functools.partial(
    pl.kernel,
    mesh=plsc.VectorSubcoreMesh(core_axis_name="c", subcore_axis_name="s"),
    out_type=(jax.ShapeDtypeStruct((B * K,), jnp.float32),
              jax.ShapeDtypeStruct((B * K,), jnp.float32)),
    scratch_types=[
        pltpu.VMEM((B,), jnp.int32),
        pltpu.VMEM((K, CH), jnp.float32),
        pltpu.VMEM((K, CH), jnp.float32),
        pltpu.VMEM_SHARED((K, NS * CH), jnp.float32),
        pltpu.VMEM_SHARED((K, NS * CH), jnp.float32),
        pltpu.VMEM((NBK * BK_U,), jnp.int32),
        pltpu.VMEM((NBK * BK_U,), jnp.int32),
        pltpu.VMEM((B * K // 16,), jnp.float32),
        pltpu.VMEM((K, U_PART_SZ), jnp.float32),
        pltpu.VMEM((K, M_PART_SZ), jnp.float32),
        pltpu.SMEM((8 + NBK,), jnp.int32),
        pltpu.SemaphoreType.DMA,
        pltpu.SemaphoreType.DMA,
        pltpu.SemaphoreType.DMA,
    ],
    compiler_params=pltpu.CompilerParams(
        needs_layout_passes=False, use_tc_tiling_on_sc=True),
)
def _sc_extract(u_idx, m_idx, ut_hbm, mt_hbm, ut_tail, mt_tail,
                u_rows, m_rows, idx_all, cb0, cb1, sp0, sp1,
                hit_uj, hit_j, stage, tail_u_buf, tail_m_buf, cnt,
                sem0, sem1, osem):
    _extract_body(u_idx, m_idx, ut_hbm, mt_hbm, ut_tail, mt_tail,
                  u_rows, m_rows, idx_all, cb0, cb1, sp0, sp1,
                  hit_uj, hit_j, stage, tail_u_buf, tail_m_buf, cnt,
                  sem0, sem1, osem)


def _dot_body(u_rows, m_rows, out_hbm, ru, rm, out_v):
    wid = lax.axis_index("s") * NC + lax.axis_index("c")
    base_e = wid * BPW * K
    pltpu.sync_copy(u_rows.at[pl.ds(base_e, BPW * K)], ru)
    pltpu.sync_copy(m_rows.at[pl.ds(base_e, BPW * K)], rm)

    lane = lax.iota(jnp.int32, 16)

    def group(g, carry):
        base = pl.multiple_of(g * 16, 16)
        row = (base + lane) * K
        acc = jnp.zeros((16,), jnp.float32)
        for k in range(K):
            u = plsc.load_gather(ru, [row + k])
            m = plsc.load_gather(rm, [row + k])
            acc = acc + u * m
        out_v[pl.ds(base, 16)] = 1.0 / (1.0 + jnp.exp(jnp.minimum(-acc, 80.0)))
        return carry

    lax.fori_loop(0, NGRP, group, 0)
    pltpu.sync_copy(out_v, out_hbm.at[wid])


@functools.partial(
    pl.kernel,
    mesh=plsc.VectorSubcoreMesh(core_axis_name="c", subcore_axis_name="s"),
    out_type=jax.ShapeDtypeStruct((NW, BPW), jnp.float32),
    scratch_types=[
        pltpu.VMEM((BPW * K,), jnp.float32),
        pltpu.VMEM((BPW * K,), jnp.float32),
        pltpu.VMEM((BPW,), jnp.float32),
    ],
    compiler_params=pltpu.CompilerParams(
        needs_layout_passes=False, use_tc_tiling_on_sc=False),
)
def _sc_dot_sigmoid(u_rows, m_rows, out_hbm, ru, rm, out_v):
    _dot_body(u_rows, m_rows, out_hbm, ru, rm, out_v)


L1_BN = 16384


def _l1_body(n_total, x_ref, o_ref):
    i = pl.program_id(0)

    @pl.when(i == 0)
    def _init():
        o_ref[0, 0] = 0.0

    x = x_ref[...]
    lane = lax.broadcasted_iota(jnp.int32, x.shape, 1)
    x = jnp.where(lane < n_total - i * L1_BN, jnp.abs(x), 0.0)
    o_ref[0, 0] += jnp.sum(x)


def _l1_sum(x_t, n_total):
    grid = (n_total + L1_BN - 1) // L1_BN
    return pl.pallas_call(
        functools.partial(_l1_body, n_total),
        grid=(grid,),
        in_specs=[pl.BlockSpec((K, L1_BN), lambda i: (0, i))],
        out_specs=pl.BlockSpec((1, 1), lambda i: (0, 0),
                               memory_space=pltpu.SMEM),
        out_shape=jax.ShapeDtypeStruct((1, 1), jnp.float32),
        compiler_params=pltpu.CompilerParams(
            dimension_semantics=("arbitrary",)),
    )(x_t)


def kernel(user, movie, user_table, user_bias_table, movie_table,
           movie_bias_table):
    del user_bias_table, movie_bias_table
    ut_t = user_table.T
    mt_t = movie_table.T
    ut_tail = user_table[U_PART:].T
    mt_tail = movie_table[M_PART:].T
    u_rows, m_rows = _sc_extract(user, movie, ut_t, mt_t, ut_tail, mt_tail)
    sig = _sc_dot_sigmoid(u_rows, m_rows).reshape(B)
    l1 = _l1_sum(ut_t, USER_N)[0, 0] + _l1_sum(mt_t, MOVIE_N)[0, 0]
    return (sig, l1)


# speedup vs baseline: 1.1116x; 1.1116x over previous
import functools

import jax
import jax.numpy as jnp
from jax import lax
from jax.experimental import pallas as pl
from jax.experimental.pallas import tpu as pltpu
from jax.experimental.pallas import tpu_sc as plsc

USER_N = 1000000
MOVIE_N = 100000
K = 32
B = 16384

NC = 2
NS = 16
NW = NC * NS
BPW = B // NW
NGRP = BPW // 16

CH = 512
NBUF = 3
NBK = 64
BK_U = 48
NBK_M = 10
BK_M = 160
SENTINEL = 0x3FFFFFFF

SLAB_U = 31232
NCH_U = SLAB_U // CH
U_TAIL = (NW * SLAB_U, 512)
U_PART = USER_N - USER_N % 128
U_PART_SZ = USER_N - U_PART

SLAB_M = 3072
NCH_M = SLAB_M // CH
M_TAILS = [(98304, 512), (98816, 512), (99328, 512), (99840, 128)]
M_PART = MOVIE_N - MOVIE_N % 128
M_PART_SZ = MOVIE_N - M_PART


def _extract_table(wid, idx_hbm, tbl_hbm, tail_hbm, tail_buf, out1d,
                   idx_all, bufs, sems, hit_uj, hit_j, stage, cnt,
                   osem, slab, n_total, nch, tails, part_lo, nbk, bk_cap):
    lane = lax.iota(jnp.int32, 16)
    lane0 = lane == 0
    lo = wid * slab
    hi = jnp.where(wid == NW - 1, n_total, lo + slab)

    pltpu.sync_copy(idx_hbm, idx_all)
    cnt[0] = 0
    cnt[1] = 0

    def zero_bk(i, carry):
        cnt[8 + i] = 0
        return carry
    lax.fori_loop(0, nbk, zero_bk, 0)

    def init(v, carry):
        hit_uj[pl.ds(v * 16, 16)] = jnp.full((16,), SENTINEL, jnp.int32)
        return carry
    lax.fori_loop(0, (nbk * bk_cap) // 16, init, 0)

    def scanv(v, carry):
        uv = idx_all[pl.ds(v * 16, 16)]
        m_init = ((uv >= lo) & (uv < hi)).astype(jnp.int32)

        @pl.when(jnp.max(m_init) > 0)
        def _():
            def bit_body(m):
                l = plsc.all_reduce_ffs(m != 0)
                one = (lane == l).astype(jnp.int32)
                uj_s = jnp.sum(uv * one)
                l_s = jnp.max(l)
                bk = (uj_s - lo) >> 9
                cb = cnt[8 + bk]
                slot = bk * bk_cap + cb
                svec = jnp.full((16,), slot, jnp.int32)
                plsc.store_scatter(hit_uj, [svec],
                                   jnp.full((16,), uj_s, jnp.int32),
                                   mask=lane0)
                plsc.store_scatter(hit_j, [svec],
                                   jnp.full((16,), v * 16 + l_s, jnp.int32),
                                   mask=lane0)
                cnt[8 + bk] = cb + 1
                return m * (1 - one)

            lax.while_loop(lambda m: jnp.max(m) > 0, bit_body, m_init)
        return carry

    lax.fori_loop(0, B // 16, scanv, 0)

    def process_chunk(buf, clo, sz):
        bk_base = ((clo - lo) >> 9) * bk_cap

        def hscan(h, carry):
            huj = hit_uj[pl.ds(bk_base + h * 16, 16)]
            hjv = hit_j[pl.ds(bk_base + h * 16, 16)]
            m_init = ((huj >= clo) & (huj < clo + sz)).astype(jnp.int32)

            @pl.when(jnp.max(m_init) > 0)
            def _():
                def bit_body(m):
                    l = plsc.all_reduce_ffs(m != 0)
                    one = (lane == l).astype(jnp.int32)
                    uj_s = jnp.sum(huj * one)
                    j_s = jnp.sum(hjv * one)
                    loc = jnp.full((16,), uj_s - clo, jnp.int32)
                    u0 = plsc.load_gather(buf, [lane, loc])
                    u1 = plsc.load_gather(buf, [lane + 16, loc])
                    c2 = cnt[1]
                    stage[pl.ds(c2 * K, 16)] = u0
                    stage[pl.ds(c2 * K + 16, 16)] = u1
                    pltpu.make_async_copy(
                        stage.at[pl.ds(c2 * K, K)],
                        out1d.at[pl.ds(j_s * K, K)],
                        osem).start()
                    cnt[1] = c2 + 1
                    return m * (1 - one)

                lax.while_loop(lambda m: jnp.max(m) > 0, bit_body, m_init)
            return carry

        lax.fori_loop(0, bk_cap // 16, hscan, 0)

    def start_window(i, b):
        for r in range(K // 8):
            pltpu.make_async_copy(
                tbl_hbm.at[pl.ds(r * 8, 8), pl.ds(lo + i * CH, CH)],
                bufs[b].at[pl.ds(r * 8, 8)], sems[b]).start()

    for b in range(NBUF):
        if b < nch:
            start_window(b, b)

    nloop = nch // NBUF

    def ring(g, carry):
        for b in range(NBUF):
            i = g * NBUF + b
            pltpu.make_async_copy(
                tbl_hbm.at[:, pl.ds(0, CH)], bufs[b], sems[b]).wait()
            process_chunk(bufs[b], lo + i * CH, CH)
            nxt = i + NBUF

            @pl.when(nxt < nch)
            def _():
                start_window(nxt, b)
        return carry

    lax.fori_loop(0, nloop, ring, 0)

    for i in range(nloop * NBUF, nch):
        b = i % NBUF
        pltpu.make_async_copy(
            tbl_hbm.at[:, pl.ds(0, CH)], bufs[b], sems[b]).wait()
        process_chunk(bufs[b], lo + i * CH, CH)

    @pl.when(wid == NW - 1)
    def _tail():
        for off, sz in tails:
            pltpu.sync_copy(tbl_hbm.at[:, pl.ds(jnp.int32(off), sz)],
                            bufs[0].at[:, pl.ds(0, sz)])
            process_chunk(bufs[0], jnp.int32(off), sz)
        pltpu.sync_copy(tail_hbm, tail_buf)
        process_chunk(tail_buf, jnp.int32(part_lo), tail_buf.shape[1])

    def drain(i, carry):
        pltpu.make_async_copy(
            out1d.at[pl.ds(0, K)], stage.at[pl.ds(0, K)], osem).wait()
        return carry
    lax.fori_loop(0, cnt[1], drain, 0)


def _extract_body(u_idx, m_idx, ut_hbm, mt_hbm, ut_tail, mt_tail,
                  u_rows, m_rows, idx_all, cb0, cb1, cb2,
                  hit_uj, hit_j, stage, tail_u_buf, tail_m_buf, cnt,
                  sem0, sem1, sem2, osem):
    wid = lax.axis_index("s") * NC + lax.axis_index("c")
    bufs = (cb0, cb1, cb2)
    sems = (sem0, sem1, sem2)
    _extract_table(wid, u_idx, ut_hbm, ut_tail, tail_u_buf, u_rows,
                   idx_all, bufs, sems, hit_uj, hit_j, stage, cnt,
                   osem, SLAB_U, USER_N, NCH_U, [U_TAIL], U_PART, NBK, BK_U)
    _extract_table(wid, m_idx, mt_hbm, mt_tail, tail_m_buf, m_rows,
                   idx_all, bufs, sems, hit_uj, hit_j, stage, cnt,
                   osem, SLAB_M, MOVIE_N, NCH_M, M_TAILS, M_PART,
                   NBK_M, BK_M)


@functools.partial(
    pl.kernel,
    mesh=plsc.VectorSubcoreMesh(core_axis_name="c", subcore_axis_name="s"),
    out_type=(jax.ShapeDtypeStruct((B * K,), jnp.float32),
              jax.ShapeDtypeStruct((B * K,), jnp.float32)),
    scratch_types=[
        pltpu.VMEM((B,), jnp.int32),
        pltpu.VMEM((K, CH), jnp.float32),
        pltpu.VMEM((K, CH), jnp.float32),
        pltpu.VMEM((K, CH), jnp.float32),
        pltpu.VMEM((NBK * BK_U,), jnp.int32),
        pltpu.VMEM((NBK * BK_U,), jnp.int32),
        pltpu.VMEM((B * K // 16,), jnp.float32),
        pltpu.VMEM((K, U_PART_SZ), jnp.float32),
        pltpu.VMEM((K, M_PART_SZ), jnp.float32),
        pltpu.SMEM((8 + NBK,), jnp.int32),
        pltpu.SemaphoreType.DMA,
        pltpu.SemaphoreType.DMA,
        pltpu.SemaphoreType.DMA,
        pltpu.SemaphoreType.DMA,
    ],
    compiler_params=pltpu.CompilerParams(
        needs_layout_passes=False, use_tc_tiling_on_sc=True),
)
def _sc_extract(u_idx, m_idx, ut_hbm, mt_hbm, ut_tail, mt_tail,
                u_rows, m_rows, idx_all, cb0, cb1, cb2,
                hit_uj, hit_j, stage, tail_u_buf, tail_m_buf, cnt,
                sem0, sem1, sem2, osem):
    _extract_body(u_idx, m_idx, ut_hbm, mt_hbm, ut_tail, mt_tail,
                  u_rows, m_rows, idx_all, cb0, cb1, cb2,
                  hit_uj, hit_j, stage, tail_u_buf, tail_m_buf, cnt,
                  sem0, sem1, sem2, osem)


def _dot_body(u_rows, m_rows, out_hbm, ru, rm, out_v):
    wid = lax.axis_index("s") * NC + lax.axis_index("c")
    base_e = wid * BPW * K
    pltpu.sync_copy(u_rows.at[pl.ds(base_e, BPW * K)], ru)
    pltpu.sync_copy(m_rows.at[pl.ds(base_e, BPW * K)], rm)

    lane = lax.iota(jnp.int32, 16)

    def group(g, carry):
        base = pl.multiple_of(g * 16, 16)
        row = (base + lane) * K
        acc = jnp.zeros((16,), jnp.float32)
        for k in range(K):
            u = plsc.load_gather(ru, [row + k])
            m = plsc.load_gather(rm, [row + k])
            acc = acc + u * m
        out_v[pl.ds(base, 16)] = 1.0 / (1.0 + jnp.exp(jnp.minimum(-acc, 80.0)))
        return carry

    lax.fori_loop(0, NGRP, group, 0)
    pltpu.sync_copy(out_v, out_hbm.at[wid])


@functools.partial(
    pl.kernel,
    mesh=plsc.VectorSubcoreMesh(core_axis_name="c", subcore_axis_name="s"),
    out_type=jax.ShapeDtypeStruct((NW, BPW), jnp.float32),
    scratch_types=[
        pltpu.VMEM((BPW * K,), jnp.float32),
        pltpu.VMEM((BPW * K,), jnp.float32),
        pltpu.VMEM((BPW,), jnp.float32),
    ],
    compiler_params=pltpu.CompilerParams(
        needs_layout_passes=False, use_tc_tiling_on_sc=False),
)
def _sc_dot_sigmoid(u_rows, m_rows, out_hbm, ru, rm, out_v):
    _dot_body(u_rows, m_rows, out_hbm, ru, rm, out_v)


L1_BN = 16384


def _l1_body(n_total, x_ref, o_ref):
    i = pl.program_id(0)

    @pl.when(i == 0)
    def _init():
        o_ref[0, 0] = 0.0

    x = x_ref[...]
    lane = lax.broadcasted_iota(jnp.int32, x.shape, 1)
    x = jnp.where(lane < n_total - i * L1_BN, jnp.abs(x), 0.0)
    o_ref[0, 0] += jnp.sum(x)


def _l1_sum(x_t, n_total):
    grid = (n_total + L1_BN - 1) // L1_BN
    return pl.pallas_call(
        functools.partial(_l1_body, n_total),
        grid=(grid,),
        in_specs=[pl.BlockSpec((K, L1_BN), lambda i: (0, i))],
        out_specs=pl.BlockSpec((1, 1), lambda i: (0, 0),
                               memory_space=pltpu.SMEM),
        out_shape=jax.ShapeDtypeStruct((1, 1), jnp.float32),
        compiler_params=pltpu.CompilerParams(
            dimension_semantics=("arbitrary",)),
    )(x_t)


def kernel(user, movie, user_table, user_bias_table, movie_table,
           movie_bias_table):
    del user_bias_table, movie_bias_table
    ut_t = user_table.T
    mt_t = movie_table.T
    ut_tail = user_table[U_PART:].T
    mt_tail = movie_table[M_PART:].T
    u_rows, m_rows = _sc_extract(user, movie, ut_t, mt_t, ut_tail, mt_tail)
    sig = _sc_dot_sigmoid(u_rows, m_rows).reshape(B)
    l1 = _l1_sum(ut_t, USER_N)[0, 0] + _l1_sum(mt_t, MOVIE_N)[0, 0]
    return (sig, l1)


# speedup vs baseline: 1.1232x; 1.0104x over previous
import functools

import jax
import jax.numpy as jnp
from jax import lax
from jax.experimental import pallas as pl
from jax.experimental.pallas import tpu as pltpu
from jax.experimental.pallas import tpu_sc as plsc

USER_N = 1000000
MOVIE_N = 100000
K = 32
B = 16384

NC = 2
NS = 16
NW = NC * NS
BPW = B // NW
NGRP = BPW // 16

CH = 512
NBUF = 3
NBK = 64
BK_U = 48
NBK_M = 10
BK_M = 160
SENTINEL = 0x3FFFFFFF

SLAB_U = 31232
NCH_U = SLAB_U // CH
U_TAIL = (NW * SLAB_U, 512)
U_PART = USER_N - USER_N % 128
U_PART_SZ = USER_N - U_PART

SLAB_M = 3072
NCH_M = SLAB_M // CH
M_TAILS = [(98304, 512), (98816, 512), (99328, 512), (99840, 128)]
M_PART = MOVIE_N - MOVIE_N % 128
M_PART_SZ = MOVIE_N - M_PART


def _extract_table(wid, idx_hbm, tbl_hbm, tail_hbm, tail_buf, out1d,
                   idx_all, bufs, sems, hit_uj, hit_j, stage, cnt,
                   osem, slab, n_total, nch, tails, part_lo, nbk, bk_cap):
    lane = lax.iota(jnp.int32, 16)
    lane0 = lane == 0
    lo = wid * slab
    hi = jnp.where(wid == NW - 1, n_total, lo + slab)

    pltpu.sync_copy(idx_hbm, idx_all)
    cnt[0] = 0
    cnt[1] = 0

    def zero_bk(i, carry):
        cnt[8 + i] = 0
        return carry
    lax.fori_loop(0, nbk, zero_bk, 0)

    def init(v, carry):
        hit_uj[pl.ds(v * 16, 16)] = jnp.full((16,), SENTINEL, jnp.int32)
        return carry
    lax.fori_loop(0, (nbk * bk_cap) // 16, init, 0)

    def start_window(i, b):
        for r in range(K // 8):
            pltpu.make_async_copy(
                tbl_hbm.at[pl.ds(r * 8, 8), pl.ds(lo + i * CH, CH)],
                bufs[b].at[pl.ds(r * 8, 8)], sems[b]).start()

    for b in range(NBUF):
        if b < nch:
            start_window(b, b)

    def scanv(v, carry):
        uv = idx_all[pl.ds(v * 16, 16)]
        m_init = ((uv >= lo) & (uv < hi)).astype(jnp.int32)

        @pl.when(jnp.max(m_init) > 0)
        def _():
            def bit_body(m):
                l = plsc.all_reduce_ffs(m != 0)
                one = (lane == l).astype(jnp.int32)
                uj_s = jnp.sum(uv * one)
                l_s = jnp.max(l)
                bk = (uj_s - lo) >> 9
                cb = cnt[8 + bk]
                slot = bk * bk_cap + cb
                svec = jnp.full((16,), slot, jnp.int32)
                plsc.store_scatter(hit_uj, [svec],
                                   jnp.full((16,), uj_s, jnp.int32),
                                   mask=lane0)
                plsc.store_scatter(hit_j, [svec],
                                   jnp.full((16,), v * 16 + l_s, jnp.int32),
                                   mask=lane0)
                cnt[8 + bk] = cb + 1
                return m * (1 - one)

            lax.while_loop(lambda m: jnp.max(m) > 0, bit_body, m_init)
        return carry

    lax.fori_loop(0, B // 16, scanv, 0)

    def process_chunk(buf, clo, sz):
        bk_base = ((clo - lo) >> 9) * bk_cap

        def hscan(h, carry):
            huj = hit_uj[pl.ds(bk_base + h * 16, 16)]
            hjv = hit_j[pl.ds(bk_base + h * 16, 16)]
            m_init = ((huj >= clo) & (huj < clo + sz)).astype(jnp.int32)

            @pl.when(jnp.max(m_init) > 0)
            def _():
                def bit_body(m):
                    l = plsc.all_reduce_ffs(m != 0)
                    one = (lane == l).astype(jnp.int32)
                    uj_s = jnp.sum(huj * one)
                    j_s = jnp.sum(hjv * one)
                    loc = jnp.full((16,), uj_s - clo, jnp.int32)
                    u0 = plsc.load_gather(buf, [lane, loc])
                    u1 = plsc.load_gather(buf, [lane + 16, loc])
                    c2 = cnt[1]
                    stage[pl.ds(c2 * K, 16)] = u0
                    stage[pl.ds(c2 * K + 16, 16)] = u1
                    pltpu.make_async_copy(
                        stage.at[pl.ds(c2 * K, K)],
                        out1d.at[pl.ds(j_s * K, K)],
                        osem).start()
                    cnt[1] = c2 + 1
                    return m * (1 - one)

                lax.while_loop(lambda m: jnp.max(m) > 0, bit_body, m_init)
            return carry

        lax.fori_loop(0, bk_cap // 16, hscan, 0)

    nloop = nch // NBUF

    def ring(g, carry):
        for b in range(NBUF):
            i = g * NBUF + b
            pltpu.make_async_copy(
                tbl_hbm.at[:, pl.ds(0, CH)], bufs[b], sems[b]).wait()
            process_chunk(bufs[b], lo + i * CH, CH)
            nxt = i + NBUF

            @pl.when(nxt < nch)
            def _():
                start_window(nxt, b)
        return carry

    lax.fori_loop(0, nloop, ring, 0)

    for i in range(nloop * NBUF, nch):
        b = i % NBUF
        pltpu.make_async_copy(
            tbl_hbm.at[:, pl.ds(0, CH)], bufs[b], sems[b]).wait()
        process_chunk(bufs[b], lo + i * CH, CH)

    @pl.when(wid == NW - 1)
    def _tail():
        for off, sz in tails:
            pltpu.sync_copy(tbl_hbm.at[:, pl.ds(jnp.int32(off), sz)],
                            bufs[0].at[:, pl.ds(0, sz)])
            process_chunk(bufs[0], jnp.int32(off), sz)
        pltpu.sync_copy(tail_hbm, tail_buf)
        process_chunk(tail_buf, jnp.int32(part_lo), tail_buf.shape[1])

    def drain(i, carry):
        pltpu.make_async_copy(
            out1d.at[pl.ds(0, K)], stage.at[pl.ds(0, K)], osem).wait()
        return carry
    lax.fori_loop(0, cnt[1], drain, 0)


def _extract_body(u_idx, m_idx, ut_hbm, mt_hbm, ut_tail, mt_tail,
                  u_rows, m_rows, idx_all, cb0, cb1, cb2,
                  hit_uj, hit_j, stage, tail_u_buf, tail_m_buf, cnt,
                  sem0, sem1, sem2, osem):
    wid = lax.axis_index("s") * NC + lax.axis_index("c")
    bufs = (cb0, cb1, cb2)
    sems = (sem0, sem1, sem2)
    _extract_table(wid, u_idx, ut_hbm, ut_tail, tail_u_buf, u_rows,
                   idx_all, bufs, sems, hit_uj, hit_j, stage, cnt,
                   osem, SLAB_U, USER_N, NCH_U, [U_TAIL], U_PART, NBK, BK_U)
    _extract_table(wid, m_idx, mt_hbm, mt_tail, tail_m_buf, m_rows,
                   idx_all, bufs, sems, hit_uj, hit_j, stage, cnt,
                   osem, SLAB_M, MOVIE_N, NCH_M, M_TAILS, M_PART,
                   NBK_M, BK_M)


@functools.partial(
    pl.kernel,
    mesh=plsc.VectorSubcoreMesh(core_axis_name="c", subcore_axis_name="s"),
    out_type=(jax.ShapeDtypeStruct((B * K,), jnp.float32),
              jax.ShapeDtypeStruct((B * K,), jnp.float32)),
    scratch_types=[
        pltpu.VMEM((B,), jnp.int32),
        pltpu.VMEM((K, CH), jnp.float32),
        pltpu.VMEM((K, CH), jnp.float32),
        pltpu.VMEM((K, CH), jnp.float32),
        pltpu.VMEM((NBK * BK_U,), jnp.int32),
        pltpu.VMEM((NBK * BK_U,), jnp.int32),
        pltpu.VMEM((B * K // 16,), jnp.float32),
        pltpu.VMEM((K, U_PART_SZ), jnp.float32),
        pltpu.VMEM((K, M_PART_SZ), jnp.float32),
        pltpu.SMEM((8 + NBK,), jnp.int32),
        pltpu.SemaphoreType.DMA,
        pltpu.SemaphoreType.DMA,
        pltpu.SemaphoreType.DMA,
        pltpu.SemaphoreType.DMA,
    ],
    compiler_params=pltpu.CompilerParams(
        needs_layout_passes=False, use_tc_tiling_on_sc=True),
)
def _sc_extract(u_idx, m_idx, ut_hbm, mt_hbm, ut_tail, mt_tail,
                u_rows, m_rows, idx_all, cb0, cb1, cb2,
                hit_uj, hit_j, stage, tail_u_buf, tail_m_buf, cnt,
                sem0, sem1, sem2, osem):
    _extract_body(u_idx, m_idx, ut_hbm, mt_hbm, ut_tail, mt_tail,
                  u_rows, m_rows, idx_all, cb0, cb1, cb2,
                  hit_uj, hit_j, stage, tail_u_buf, tail_m_buf, cnt,
                  sem0, sem1, sem2, osem)


def _dot_body(u_rows, m_rows, out_hbm, ru, rm, out_v):
    wid = lax.axis_index("s") * NC + lax.axis_index("c")
    base_e = wid * BPW * K
    pltpu.sync_copy(u_rows.at[pl.ds(base_e, BPW * K)], ru)
    pltpu.sync_copy(m_rows.at[pl.ds(base_e, BPW * K)], rm)

    lane = lax.iota(jnp.int32, 16)

    def group(g, carry):
        base = pl.multiple_of(g * 16, 16)
        row = (base + lane) * K
        acc = jnp.zeros((16,), jnp.float32)
        for k in range(K):
            u = plsc.load_gather(ru, [row + k])
            m = plsc.load_gather(rm, [row + k])
            acc = acc + u * m
        out_v[pl.ds(base, 16)] = 1.0 / (1.0 + jnp.exp(jnp.minimum(-acc, 80.0)))
        return carry

    lax.fori_loop(0, NGRP, group, 0)
    pltpu.sync_copy(out_v, out_hbm.at[wid])


@functools.partial(
    pl.kernel,
    mesh=plsc.VectorSubcoreMesh(core_axis_name="c", subcore_axis_name="s"),
    out_type=jax.ShapeDtypeStruct((NW, BPW), jnp.float32),
    scratch_types=[
        pltpu.VMEM((BPW * K,), jnp.float32),
        pltpu.VMEM((BPW * K,), jnp.float32),
        pltpu.VMEM((BPW,), jnp.float32),
    ],
    compiler_params=pltpu.CompilerParams(
        needs_layout_passes=False, use_tc_tiling_on_sc=False),
)
def _sc_dot_sigmoid(u_rows, m_rows, out_hbm, ru, rm, out_v):
    _dot_body(u_rows, m_rows, out_hbm, ru, rm, out_v)


L1_BN = 16384


def _l1_body(n_total, x_ref, o_ref):
    i = pl.program_id(0)

    @pl.when(i == 0)
    def _init():
        o_ref[0, 0] = 0.0

    x = x_ref[...]
    lane = lax.broadcasted_iota(jnp.int32, x.shape, 1)
    x = jnp.where(lane < n_total - i * L1_BN, jnp.abs(x), 0.0)
    o_ref[0, 0] += jnp.sum(x)


def _l1_sum(x_t, n_total):
    grid = (n_total + L1_BN - 1) // L1_BN
    return pl.pallas_call(
        functools.partial(_l1_body, n_total),
        grid=(grid,),
        in_specs=[pl.BlockSpec((K, L1_BN), lambda i: (0, i))],
        out_specs=pl.BlockSpec((1, 1), lambda i: (0, 0),
                               memory_space=pltpu.SMEM),
        out_shape=jax.ShapeDtypeStruct((1, 1), jnp.float32),
        compiler_params=pltpu.CompilerParams(
            dimension_semantics=("arbitrary",)),
    )(x_t)


def kernel(user, movie, user_table, user_bias_table, movie_table,
           movie_bias_table):
    del user_bias_table, movie_bias_table
    ut_t = user_table.T
    mt_t = movie_table.T
    ut_tail = user_table[U_PART:].T
    mt_tail = movie_table[M_PART:].T
    u_rows, m_rows = _sc_extract(user, movie, ut_t, mt_t, ut_tail, mt_tail)
    sig = _sc_dot_sigmoid(u_rows, m_rows).reshape(B)
    l1 = _l1_sum(ut_t, USER_N)[0, 0] + _l1_sum(mt_t, MOVIE_N)[0, 0]
    return (sig, l1)
